# baseline (speedup 1.0000x reference)
import functools

import jax
import jax.numpy as jnp
from jax import lax
from jax.experimental import pallas as pl
from jax.experimental.pallas import tpu as pltpu
from jax.experimental.pallas import tpu_sc as plsc

NUM_EMB = 1_000_000
DIM = 64
BATCH = 4096
SEQ_LEN = 200

NC = 2
NS = 16
NW = NC * NS
BW = BATCH // NW
L = 16


VCHUNK = 128
NCHUNK = NUM_EMB // VCHUNK
VTAIL = NUM_EMB - NCHUNK * VCHUNK
TP = 131


@functools.partial(
    pl.kernel,
    mesh=plsc.VectorSubcoreMesh(core_axis_name="c", subcore_axis_name="s"),
    compiler_params=pltpu.CompilerParams(
        use_tc_tiling_on_sc=True, needs_layout_passes=False
    ),
    out_type=jax.ShapeDtypeStruct((NUM_EMB // 2, 2 * DIM), jnp.float32),
    scratch_types=[
        pltpu.VMEM((2, 8, 8, VCHUNK), jnp.float32),
        pltpu.VMEM((2, DIM, TP), jnp.float32),
        pltpu.VMEM((VTAIL // 2, 2 * DIM), jnp.float32),
        pltpu.SemaphoreType.DMA,
        pltpu.SemaphoreType.DMA,
    ],
)
def _transpose_w(wt_hbm, tail_hbm, out_hbm, in_v, tr_v, tail_v, isem, osem):
    wid = lax.axis_index("s") * NC + lax.axis_index("c")

    def v0_of(i):
        return (wid + i * NW) * VCHUNK

    def fire_loads(i, b):
        v0 = v0_of(i)
        for j8 in range(8):
            pltpu.async_copy(
                wt_hbm.at[pl.ds(j8 * 8, 8), pl.ds(v0, VCHUNK)],
                in_v.at[b, j8],
                isem,
            )

    def wait_loads(i, b):
        v0 = v0_of(i)
        for j8 in range(8):
            pltpu.make_async_copy(
                wt_hbm.at[pl.ds(j8 * 8, 8), pl.ds(v0, VCHUNK)],
                in_v.at[b, j8],
                isem,
            ).wait()

    def transpose(b):
        tf = tr_v.at[b]
        half = (lax.iota(jnp.int32, L) % 2) * DIM

        def jbody(j8, carry):
            src = in_v.at[b, j8]
            for u in range(8):
                j = j8 * 8 + u
                colv = half + j
                for g in range(VCHUNK // L):
                    rowv = (lax.iota(jnp.int32, L) + g * L) // 2
                    plsc.store_scatter(
                        tf, [rowv, colv], src[u, pl.ds(g * L, L)]
                    )
            return carry

        lax.fori_loop(0, 8, jbody, 0)

    niter = (NCHUNK - wid + NW - 1) // NW

    fire_loads(0, 0)

    def body(i, carry):
        b = lax.rem(i, 2)
        nb = lax.rem(i + 1, 2)
        wait_loads(i, b)

        @pl.when(i + 1 < niter)
        def _():
            fire_loads(i + 1, nb)

        @pl.when(i >= 2)
        def _():
            pltpu.make_async_copy(
                tr_v.at[b, pl.ds(0, DIM), pl.ds(0, 2 * DIM)],
                out_hbm.at[pl.ds(0, DIM)],
                osem,
            ).wait()

        transpose(b)
        pltpu.async_copy(
            tr_v.at[b, pl.ds(0, DIM), pl.ds(0, 2 * DIM)],
            out_hbm.at[pl.ds(pl.multiple_of(v0_of(i) // 2, DIM), DIM)],
            osem,
        )
        return carry

    lax.fori_loop(0, niter, body, 0)
    for b in range(2):
        pltpu.make_async_copy(
            tr_v.at[b, pl.ds(0, DIM), pl.ds(0, 2 * DIM)],
            out_hbm.at[pl.ds(0, DIM)],
            osem,
        ).wait()

    @pl.when(wid == 0)
    def _():
        pltpu.sync_copy(tail_hbm, tail_v)
        pltpu.sync_copy(
            tail_v, out_hbm.at[pl.ds(NCHUNK * VCHUNK // 2, VTAIL // 2)]
        )


@functools.partial(
    pl.kernel,
    mesh=plsc.VectorSubcoreMesh(core_axis_name="c", subcore_axis_name="s"),
    compiler_params=pltpu.CompilerParams(
        use_tc_tiling_on_sc=False, needs_layout_passes=False
    ),
    out_type=jax.ShapeDtypeStruct((SEQ_LEN, DIM, BATCH), jnp.float32),
    scratch_types=[
        pltpu.VMEM((SEQ_LEN, BW), jnp.int32),
        pltpu.VMEM((2, BW, DIM), jnp.float32),
        pltpu.VMEM((2, DIM, BW + 1), jnp.float32),
        pltpu.SemaphoreType.DMA,
        pltpu.SemaphoreType.DMA,
    ],
)
def _gather(w_hbm, idx_hbm, out_hbm, idx_v, rows_v, tr_v, gsem, osem):
    wid = lax.axis_index("s") * NC + lax.axis_index("c")
    c0 = wid * BW

    pltpu.sync_copy(idx_hbm.at[pl.ds(0, SEQ_LEN), pl.ds(c0, BW)], idx_v)

    def fire_gather(s, b):
        pltpu.async_copy(w_hbm.at[idx_v.at[s]], rows_v.at[b], gsem)

    fire_gather(0, 0)
    fire_gather(1, 1)

    def transpose(b):
        rf = rows_v.at[b]
        tf = tr_v.at[b]

        def rbody(r8, carry):
            for u in range(8):
                r = r8 * 8 + u
                colv = jnp.broadcast_to(r, (L,))
                for q in range(DIM // L):
                    rowv = lax.iota(jnp.int32, L) + q * L
                    plsc.store_scatter(tf, [rowv, colv], rf[r, pl.ds(q * L, L)])
            return carry

        lax.fori_loop(0, BW // 8, rbody, 0)

    def body(sblk, carry):
        for b in range(2):
            s = sblk * 2 + b
            pltpu.make_async_copy(
                w_hbm.at[idx_v.at[s]], rows_v.at[b], gsem
            ).wait()

            @pl.when(sblk >= 1)
            def _():
                pltpu.make_async_copy(
                    tr_v.at[b, pl.ds(0, DIM), pl.ds(0, BW)],
                    out_hbm.at[0, pl.ds(0, DIM), pl.ds(c0, BW)],
                    osem,
                ).wait()

            transpose(b)
            pltpu.async_copy(
                tr_v.at[b, pl.ds(0, DIM), pl.ds(0, BW)],
                out_hbm.at[s, pl.ds(0, DIM), pl.ds(c0, BW)],
                osem,
            )

            @pl.when(s + 2 < SEQ_LEN)
            def _():
                fire_gather(s + 2, b)
        return carry

    lax.fori_loop(0, SEQ_LEN // 2, body, 0)
    for b in range(2):
        pltpu.make_async_copy(
            tr_v.at[b, pl.ds(0, DIM), pl.ds(0, BW)],
            out_hbm.at[0, pl.ds(0, DIM), pl.ds(c0, BW)],
            osem,
        ).wait()


def kernel(token_ids, W):
    tail = W[NCHUNK * VCHUNK :].reshape(VTAIL // 2, 2 * DIM)
    w_rm = _transpose_w(W.T, tail)
    w_lin = w_rm.reshape(NUM_EMB, DIM)
    tok_t = token_ids.astype(jnp.int32).T
    out = _gather(w_lin, tok_t)
    return out.transpose(2, 0, 1)


# speedup vs baseline: 1.0021x; 1.0021x over previous
import functools

import jax
import jax.numpy as jnp
from jax import lax
from jax.experimental import pallas as pl
from jax.experimental.pallas import tpu as pltpu
from jax.experimental.pallas import tpu_sc as plsc

NUM_EMB = 1_000_000
DIM = 64
BATCH = 4096
SEQ_LEN = 200

NC = 2
NS = 16
NW = NC * NS
BW = BATCH // NW
L = 16


VCHUNK = 128
NCHUNK = NUM_EMB // VCHUNK
VTAIL = NUM_EMB - NCHUNK * VCHUNK
TP = 131


@functools.partial(
    pl.kernel,
    mesh=plsc.VectorSubcoreMesh(core_axis_name="c", subcore_axis_name="s"),
    compiler_params=pltpu.CompilerParams(
        use_tc_tiling_on_sc=True, needs_layout_passes=False
    ),
    out_type=jax.ShapeDtypeStruct((NUM_EMB // 2, 2 * DIM), jnp.float32),
    scratch_types=[
        pltpu.VMEM((2, 8, 8, VCHUNK), jnp.float32),
        pltpu.VMEM((2, DIM, TP), jnp.float32),
        pltpu.VMEM((VTAIL // 2, 2 * DIM), jnp.float32),
        pltpu.SemaphoreType.DMA,
        pltpu.SemaphoreType.DMA,
    ],
)
def _transpose_w(wt_hbm, tail_hbm, out_hbm, in_v, tr_v, tail_v, isem, osem):
    wid = lax.axis_index("s") * NC + lax.axis_index("c")

    def v0_of(i):
        return (wid + i * NW) * VCHUNK

    def fire_loads(i, b):
        v0 = v0_of(i)
        for j8 in range(8):
            pltpu.async_copy(
                wt_hbm.at[pl.ds(j8 * 8, 8), pl.ds(v0, VCHUNK)],
                in_v.at[b, j8],
                isem,
            )

    def wait_loads(i, b):
        v0 = v0_of(i)
        for j8 in range(8):
            pltpu.make_async_copy(
                wt_hbm.at[pl.ds(j8 * 8, 8), pl.ds(v0, VCHUNK)],
                in_v.at[b, j8],
                isem,
            ).wait()

    def transpose(b):
        tf = tr_v.at[b]
        iota = lax.iota(jnp.int32, L)
        iota2 = iota >> 1
        half = (iota & 1) << 6

        def jbody(j8, carry):
            src = in_v.at[b, j8]
            for u in range(8):
                j = j8 * 8 + u
                colv = half + j
                for g in range(VCHUNK // L):
                    rowv = iota2 + g * (L // 2)
                    plsc.store_scatter(
                        tf, [rowv, colv], src[u, pl.ds(g * L, L)]
                    )
            return carry

        lax.fori_loop(0, 8, jbody, 0)

    niter = (NCHUNK - wid + NW - 1) // NW

    fire_loads(0, 0)

    def body(i, carry):
        b = lax.rem(i, 2)
        nb = lax.rem(i + 1, 2)
        wait_loads(i, b)

        @pl.when(i + 1 < niter)
        def _():
            fire_loads(i + 1, nb)

        @pl.when(i >= 2)
        def _():
            pltpu.make_async_copy(
                tr_v.at[b, pl.ds(0, DIM), pl.ds(0, 2 * DIM)],
                out_hbm.at[pl.ds(0, DIM)],
                osem,
            ).wait()

        transpose(b)
        pltpu.async_copy(
            tr_v.at[b, pl.ds(0, DIM), pl.ds(0, 2 * DIM)],
            out_hbm.at[pl.ds(pl.multiple_of(v0_of(i) // 2, DIM), DIM)],
            osem,
        )
        return carry

    lax.fori_loop(0, niter, body, 0)
    for b in range(2):
        pltpu.make_async_copy(
            tr_v.at[b, pl.ds(0, DIM), pl.ds(0, 2 * DIM)],
            out_hbm.at[pl.ds(0, DIM)],
            osem,
        ).wait()

    @pl.when(wid == 0)
    def _():
        pltpu.sync_copy(tail_hbm, tail_v)
        pltpu.sync_copy(
            tail_v, out_hbm.at[pl.ds(NCHUNK * VCHUNK // 2, VTAIL // 2)]
        )


@functools.partial(
    pl.kernel,
    mesh=plsc.VectorSubcoreMesh(core_axis_name="c", subcore_axis_name="s"),
    compiler_params=pltpu.CompilerParams(
        use_tc_tiling_on_sc=False, needs_layout_passes=False
    ),
    out_type=jax.ShapeDtypeStruct((SEQ_LEN, DIM, BATCH), jnp.float32),
    scratch_types=[
        pltpu.VMEM((SEQ_LEN, BW), jnp.int32),
        pltpu.VMEM((2, BW, DIM), jnp.float32),
        pltpu.VMEM((2, DIM, BW + 1), jnp.float32),
        pltpu.SemaphoreType.DMA,
        pltpu.SemaphoreType.DMA,
    ],
)
def _gather(w_hbm, idx_hbm, out_hbm, idx_v, rows_v, tr_v, gsem, osem):
    wid = lax.axis_index("s") * NC + lax.axis_index("c")
    c0 = wid * BW

    pltpu.sync_copy(idx_hbm.at[pl.ds(0, SEQ_LEN), pl.ds(c0, BW)], idx_v)

    def fire_gather(s, b):
        pltpu.async_copy(w_hbm.at[idx_v.at[s]], rows_v.at[b], gsem)

    fire_gather(0, 0)
    fire_gather(1, 1)

    def transpose(b):
        rf = rows_v.at[b]
        tf = tr_v.at[b]

        def rbody(r8, carry):
            for u in range(8):
                r = r8 * 8 + u
                colv = jnp.broadcast_to(r, (L,))
                for q in range(DIM // L):
                    rowv = lax.iota(jnp.int32, L) + q * L
                    plsc.store_scatter(tf, [rowv, colv], rf[r, pl.ds(q * L, L)])
            return carry

        lax.fori_loop(0, BW // 8, rbody, 0)

    def body(sblk, carry):
        for b in range(2):
            s = sblk * 2 + b
            pltpu.make_async_copy(
                w_hbm.at[idx_v.at[s]], rows_v.at[b], gsem
            ).wait()

            @pl.when(sblk >= 1)
            def _():
                pltpu.make_async_copy(
                    tr_v.at[b, pl.ds(0, DIM), pl.ds(0, BW)],
                    out_hbm.at[0, pl.ds(0, DIM), pl.ds(c0, BW)],
                    osem,
                ).wait()

            transpose(b)
            pltpu.async_copy(
                tr_v.at[b, pl.ds(0, DIM), pl.ds(0, BW)],
                out_hbm.at[s, pl.ds(0, DIM), pl.ds(c0, BW)],
                osem,
            )

            @pl.when(s + 2 < SEQ_LEN)
            def _():
                fire_gather(s + 2, b)
        return carry

    lax.fori_loop(0, SEQ_LEN // 2, body, 0)
    for b in range(2):
        pltpu.make_async_copy(
            tr_v.at[b, pl.ds(0, DIM), pl.ds(0, BW)],
            out_hbm.at[0, pl.ds(0, DIM), pl.ds(c0, BW)],
            osem,
        ).wait()


def kernel(token_ids, W):
    tail = W[NCHUNK * VCHUNK :].reshape(VTAIL // 2, 2 * DIM)
    w_rm = _transpose_w(W.T, tail)
    w_lin = w_rm.reshape(NUM_EMB, DIM)
    tok_t = token_ids.astype(jnp.int32).T
    out = _gather(w_lin, tok_t)
    return out.transpose(2, 0, 1)


# speedup vs baseline: 1.0065x; 1.0045x over previous
import functools

import jax
import jax.numpy as jnp
from jax import lax
from jax.experimental import pallas as pl
from jax.experimental.pallas import tpu as pltpu
from jax.experimental.pallas import tpu_sc as plsc

NUM_EMB = 1_000_000
DIM = 64
BATCH = 4096
SEQ_LEN = 200

NC = 2
NS = 16
NW = NC * NS
BW = BATCH // NW
L = 16


VCHUNK = 128
NCHUNK = NUM_EMB // VCHUNK
VTAIL = NUM_EMB - NCHUNK * VCHUNK
TP = 131


@functools.partial(
    pl.kernel,
    mesh=plsc.VectorSubcoreMesh(core_axis_name="c", subcore_axis_name="s"),
    compiler_params=pltpu.CompilerParams(
        use_tc_tiling_on_sc=True, needs_layout_passes=False
    ),
    out_type=jax.ShapeDtypeStruct((NUM_EMB // 2, 2 * DIM), jnp.float32),
    scratch_types=[
        pltpu.VMEM((2, DIM, VCHUNK), jnp.float32),
        pltpu.VMEM((2, DIM, TP), jnp.float32),
        pltpu.VMEM((VTAIL // 2, 2 * DIM), jnp.float32),
        pltpu.SemaphoreType.DMA,
        pltpu.SemaphoreType.DMA,
    ],
)
def _transpose_w(wt_hbm, tail_hbm, out_hbm, in_v, tr_v, tail_v, isem, osem):
    wid = lax.axis_index("s") * NC + lax.axis_index("c")

    def v0_of(i):
        return (wid + i * NW) * VCHUNK

    def fire_loads(i, b):
        v0 = v0_of(i)
        pltpu.async_copy(
            wt_hbm.at[pl.ds(0, DIM), pl.ds(v0, VCHUNK)],
            in_v.at[b],
            isem,
        )

    def wait_loads(i, b):
        v0 = v0_of(i)
        pltpu.make_async_copy(
            wt_hbm.at[pl.ds(0, DIM), pl.ds(v0, VCHUNK)],
            in_v.at[b],
            isem,
        ).wait()

    def transpose(b):
        tf = tr_v.at[b]
        iota = lax.iota(jnp.int32, L)
        iota2 = iota >> 1
        half = (iota & 1) << 6

        def jbody(j8, carry):
            src = in_v.at[b]
            for u in range(8):
                j = j8 * 8 + u
                colv = half + j
                for g in range(VCHUNK // L):
                    rowv = iota2 + g * (L // 2)
                    plsc.store_scatter(
                        tf, [rowv, colv], src[j, pl.ds(g * L, L)]
                    )
            return carry

        lax.fori_loop(0, 8, jbody, 0)

    niter = (NCHUNK - wid + NW - 1) // NW

    fire_loads(0, 0)

    def body(i, carry):
        b = lax.rem(i, 2)
        nb = lax.rem(i + 1, 2)
        wait_loads(i, b)

        @pl.when(i + 1 < niter)
        def _():
            fire_loads(i + 1, nb)

        @pl.when(i >= 2)
        def _():
            pltpu.make_async_copy(
                tr_v.at[b, pl.ds(0, DIM), pl.ds(0, 2 * DIM)],
                out_hbm.at[pl.ds(0, DIM)],
                osem,
            ).wait()

        transpose(b)
        pltpu.async_copy(
            tr_v.at[b, pl.ds(0, DIM), pl.ds(0, 2 * DIM)],
            out_hbm.at[pl.ds(pl.multiple_of(v0_of(i) // 2, DIM), DIM)],
            osem,
        )
        return carry

    lax.fori_loop(0, niter, body, 0)
    for b in range(2):
        pltpu.make_async_copy(
            tr_v.at[b, pl.ds(0, DIM), pl.ds(0, 2 * DIM)],
            out_hbm.at[pl.ds(0, DIM)],
            osem,
        ).wait()

    @pl.when(wid == 0)
    def _():
        pltpu.sync_copy(tail_hbm, tail_v)
        pltpu.sync_copy(
            tail_v, out_hbm.at[pl.ds(NCHUNK * VCHUNK // 2, VTAIL // 2)]
        )


@functools.partial(
    pl.kernel,
    mesh=plsc.VectorSubcoreMesh(core_axis_name="c", subcore_axis_name="s"),
    compiler_params=pltpu.CompilerParams(
        use_tc_tiling_on_sc=False, needs_layout_passes=False
    ),
    out_type=jax.ShapeDtypeStruct((SEQ_LEN, DIM, BATCH), jnp.float32),
    scratch_types=[
        pltpu.VMEM((SEQ_LEN, BW), jnp.int32),
        pltpu.VMEM((2, BW, DIM), jnp.float32),
        pltpu.VMEM((2, DIM, BW + 1), jnp.float32),
        pltpu.SemaphoreType.DMA,
        pltpu.SemaphoreType.DMA,
    ],
)
def _gather(w_hbm, idx_hbm, out_hbm, idx_v, rows_v, tr_v, gsem, osem):
    wid = lax.axis_index("s") * NC + lax.axis_index("c")
    c0 = wid * BW

    pltpu.sync_copy(idx_hbm.at[pl.ds(0, SEQ_LEN), pl.ds(c0, BW)], idx_v)

    def fire_gather(s, b):
        pltpu.async_copy(w_hbm.at[idx_v.at[s]], rows_v.at[b], gsem)

    fire_gather(0, 0)
    fire_gather(1, 1)

    def transpose(b):
        rf = rows_v.at[b]
        tf = tr_v.at[b]

        def rbody(r8, carry):
            for u in range(8):
                r = r8 * 8 + u
                colv = jnp.broadcast_to(r, (L,))
                for q in range(DIM // L):
                    rowv = lax.iota(jnp.int32, L) + q * L
                    plsc.store_scatter(tf, [rowv, colv], rf[r, pl.ds(q * L, L)])
            return carry

        lax.fori_loop(0, BW // 8, rbody, 0)

    def body(sblk, carry):
        for b in range(2):
            s = sblk * 2 + b
            pltpu.make_async_copy(
                w_hbm.at[idx_v.at[s]], rows_v.at[b], gsem
            ).wait()

            @pl.when(sblk >= 1)
            def _():
                pltpu.make_async_copy(
                    tr_v.at[b, pl.ds(0, DIM), pl.ds(0, BW)],
                    out_hbm.at[0, pl.ds(0, DIM), pl.ds(c0, BW)],
                    osem,
                ).wait()

            transpose(b)
            pltpu.async_copy(
                tr_v.at[b, pl.ds(0, DIM), pl.ds(0, BW)],
                out_hbm.at[s, pl.ds(0, DIM), pl.ds(c0, BW)],
                osem,
            )

            @pl.when(s + 2 < SEQ_LEN)
            def _():
                fire_gather(s + 2, b)
        return carry

    lax.fori_loop(0, SEQ_LEN // 2, body, 0)
    for b in range(2):
        pltpu.make_async_copy(
            tr_v.at[b, pl.ds(0, DIM), pl.ds(0, BW)],
            out_hbm.at[0, pl.ds(0, DIM), pl.ds(c0, BW)],
            osem,
        ).wait()


def kernel(token_ids, W):
    tail = W[NCHUNK * VCHUNK :].reshape(VTAIL // 2, 2 * DIM)
    w_rm = _transpose_w(W.T, tail)
    w_lin = w_rm.reshape(NUM_EMB, DIM)
    tok_t = token_ids.astype(jnp.int32).T
    out = _gather(w_lin, tok_t)
    return out.transpose(2, 0, 1)


# speedup vs baseline: 1.5111x; 1.5013x over previous
import functools

import jax
import jax.numpy as jnp
from jax import lax
from jax.experimental import pallas as pl
from jax.experimental.pallas import tpu as pltpu
from jax.experimental.pallas import tpu_sc as plsc

NUM_EMB = 1_000_000
DIM = 64
BATCH = 4096
SEQ_LEN = 200

NC = 2
NS = 16
NW = NC * NS
BW = BATCH // NW
L = 16


@functools.partial(
    pl.kernel,
    mesh=plsc.VectorSubcoreMesh(core_axis_name="c", subcore_axis_name="s"),
    compiler_params=pltpu.CompilerParams(
        use_tc_tiling_on_sc=False, needs_layout_passes=False
    ),
    out_type=jax.ShapeDtypeStruct((SEQ_LEN, DIM, BATCH), jnp.float32),
    scratch_types=[
        pltpu.VMEM((SEQ_LEN, BW), jnp.int32),
        pltpu.VMEM((2, BW, DIM), jnp.float32),
        pltpu.VMEM((2, DIM, BW + 1), jnp.float32),
        pltpu.SemaphoreType.DMA,
        pltpu.SemaphoreType.DMA,
    ],
)
def _gather(w_hbm, idx_hbm, out_hbm, idx_v, rows_v, tr_v, gsem, osem):
    wid = lax.axis_index("s") * NC + lax.axis_index("c")
    c0 = wid * BW

    pltpu.sync_copy(idx_hbm.at[pl.ds(0, SEQ_LEN), pl.ds(c0, BW)], idx_v)

    def fire_gather(s, b):
        pltpu.async_copy(w_hbm.at[idx_v.at[s]], rows_v.at[b], gsem)

    fire_gather(0, 0)
    fire_gather(1, 1)

    def transpose(b):
        rf = rows_v.at[b]
        tf = tr_v.at[b]

        def rbody(r8, carry):
            for u in range(8):
                r = r8 * 8 + u
                colv = jnp.broadcast_to(r, (L,))
                for q in range(DIM // L):
                    rowv = lax.iota(jnp.int32, L) + q * L
                    plsc.store_scatter(tf, [rowv, colv], rf[r, pl.ds(q * L, L)])
            return carry

        lax.fori_loop(0, BW // 8, rbody, 0)

    def body(sblk, carry):
        for b in range(2):
            s = sblk * 2 + b
            pltpu.make_async_copy(
                w_hbm.at[idx_v.at[s]], rows_v.at[b], gsem
            ).wait()

            @pl.when(sblk >= 1)
            def _():
                pltpu.make_async_copy(
                    tr_v.at[b, pl.ds(0, DIM), pl.ds(0, BW)],
                    out_hbm.at[0, pl.ds(0, DIM), pl.ds(c0, BW)],
                    osem,
                ).wait()

            transpose(b)
            pltpu.async_copy(
                tr_v.at[b, pl.ds(0, DIM), pl.ds(0, BW)],
                out_hbm.at[s, pl.ds(0, DIM), pl.ds(c0, BW)],
                osem,
            )

            @pl.when(s + 2 < SEQ_LEN)
            def _():
                fire_gather(s + 2, b)
        return carry

    lax.fori_loop(0, SEQ_LEN // 2, body, 0)
    for b in range(2):
        pltpu.make_async_copy(
            tr_v.at[b, pl.ds(0, DIM), pl.ds(0, BW)],
            out_hbm.at[0, pl.ds(0, DIM), pl.ds(c0, BW)],
            osem,
        ).wait()


def kernel(token_ids, W):
    tok_t = token_ids.astype(jnp.int32).T
    out = _gather(W, tok_t)
    return out.transpose(2, 0, 1)


# speedup vs baseline: 1.5113x; 1.0001x over previous
import functools

import jax
import jax.numpy as jnp
from jax import lax
from jax.experimental import pallas as pl
from jax.experimental.pallas import tpu as pltpu
from jax.experimental.pallas import tpu_sc as plsc

NUM_EMB = 1_000_000
DIM = 64
BATCH = 4096
SEQ_LEN = 200

NC = 2
NS = 16
NW = NC * NS
BW = BATCH // NW
L = 16


@functools.partial(
    pl.kernel,
    mesh=plsc.VectorSubcoreMesh(core_axis_name="c", subcore_axis_name="s"),
    compiler_params=pltpu.CompilerParams(
        use_tc_tiling_on_sc=False, needs_layout_passes=False
    ),
    out_type=jax.ShapeDtypeStruct((SEQ_LEN, DIM, BATCH), jnp.float32),
    scratch_types=[
        pltpu.VMEM((SEQ_LEN, BW), jnp.int32),
        pltpu.VMEM((2, BW, DIM), jnp.float32),
        pltpu.VMEM((2, DIM, BW + 1), jnp.float32),
        pltpu.SemaphoreType.DMA,
        pltpu.SemaphoreType.DMA,
    ],
)
def _gather(w_hbm, idx_hbm, out_hbm, idx_v, rows_v, tr_v, gsem, osem):
    wid = lax.axis_index("s") * NC + lax.axis_index("c")
    c0 = wid * BW

    pltpu.sync_copy(idx_hbm.at[pl.ds(0, SEQ_LEN), pl.ds(c0, BW)], idx_v)

    def fire_gather(s, b):
        pltpu.async_copy(w_hbm.at[idx_v.at[s]], rows_v.at[b], gsem)

    fire_gather(0, 0)
    fire_gather(1, 1)

    def transpose(b):
        rf = rows_v.at[b]
        tf = tr_v.at[b]

        def rbody(r16, carry):
            for u in range(16):
                r = r16 * 16 + u
                colv = jnp.broadcast_to(r, (L,))
                for q in range(DIM // L):
                    rowv = lax.iota(jnp.int32, L) + q * L
                    plsc.store_scatter(tf, [rowv, colv], rf[r, pl.ds(q * L, L)])
            return carry

        lax.fori_loop(0, BW // 16, rbody, 0)

    def body(sblk, carry):
        for b in range(2):
            s = sblk * 2 + b
            pltpu.make_async_copy(
                w_hbm.at[idx_v.at[s]], rows_v.at[b], gsem
            ).wait()

            @pl.when(sblk >= 1)
            def _():
                pltpu.make_async_copy(
                    tr_v.at[b, pl.ds(0, DIM), pl.ds(0, BW)],
                    out_hbm.at[0, pl.ds(0, DIM), pl.ds(c0, BW)],
                    osem,
                ).wait()

            transpose(b)
            pltpu.async_copy(
                tr_v.at[b, pl.ds(0, DIM), pl.ds(0, BW)],
                out_hbm.at[s, pl.ds(0, DIM), pl.ds(c0, BW)],
                osem,
            )

            @pl.when(s + 2 < SEQ_LEN)
            def _():
                fire_gather(s + 2, b)
        return carry

    lax.fori_loop(0, SEQ_LEN // 2, body, 0)
    for b in range(2):
        pltpu.make_async_copy(
            tr_v.at[b, pl.ds(0, DIM), pl.ds(0, BW)],
            out_hbm.at[0, pl.ds(0, DIM), pl.ds(c0, BW)],
            osem,
        ).wait()


def kernel(token_ids, W):
    tok_t = token_ids.astype(jnp.int32).T
    out = _gather(W, tok_t)
    return out.transpose(2, 0, 1)
